# baseline (speedup 1.0000x reference)
import functools

import jax
import jax.numpy as jnp
from jax import lax
from jax.experimental import pallas as pl
from jax.experimental.pallas import tpu as pltpu
from jax.experimental.pallas import tpu_sc as plsc

VOCAB = 100000
EMB = 64
BATCH = 4096
CTX = 20

NUM_CORES = 2
NUM_SUBCORES = 16
NW = NUM_CORES * NUM_SUBCORES
BAGS_PER_W = BATCH // NW
CB = 32
NCHUNK = BAGS_PER_W // CB
ROWS_PER_CHUNK = CB * CTX
LANES = 16
EMB_PAD = 128


def _sc_bag_mean(bags_flat, bag_W_pad):
    mesh = plsc.VectorSubcoreMesh(core_axis_name="c", subcore_axis_name="s")

    @functools.partial(
        pl.kernel,
        mesh=mesh,
        out_type=jax.ShapeDtypeStruct((BATCH, EMB), jnp.float32),
        scratch_types=[
            pltpu.VMEM((ROWS_PER_CHUNK,), jnp.int32),
            pltpu.VMEM((ROWS_PER_CHUNK, EMB_PAD), jnp.float32),
            pltpu.VMEM((CB, EMB), jnp.float32),
            pltpu.SemaphoreType.DMA,
        ],
    )
    def k(idx_hbm, table_hbm, out_hbm, idx_v, rows_v, acc_v, sem):
        wid = lax.axis_index("s") * NUM_CORES + lax.axis_index("c")
        idx_base = wid * (BAGS_PER_W * CTX)
        out_base = wid * BAGS_PER_W
        for i in range(NCHUNK):
            pltpu.sync_copy(
                idx_hbm.at[pl.ds(idx_base + i * ROWS_PER_CHUNK, ROWS_PER_CHUNK)],
                idx_v,
            )
            pltpu.async_copy(table_hbm.at[idx_v], rows_v, sem).wait()

            @pl.loop(0, CB)
            def _(w):
                row0 = w * CTX
                for c in range(0, EMB, LANES):
                    s = rows_v[pl.ds(row0, 1), pl.ds(c, LANES)]
                    for r in range(1, CTX):
                        s = s + rows_v[pl.ds(row0 + r, 1), pl.ds(c, LANES)]
                    acc_v[pl.ds(w, 1), pl.ds(c, LANES)] = s * (1.0 / CTX)

            pltpu.sync_copy(acc_v, out_hbm.at[pl.ds(out_base + i * CB, CB)])

    return k(bags_flat, bag_W_pad)


_VT = 400
_NV = VOCAB // _VT
_NBUF = 4


def _tc_body(avg_ref, tag_ref, out_ref, *scratch):
    bufs = scratch[:_NBUF]
    sems = scratch[_NBUF:]
    s = pl.program_id(0)

    a = avg_ref[...].astype(jnp.bfloat16)
    t = tag_ref[...].astype(jnp.bfloat16)

    def out_copy(kk, ss):
        return pltpu.make_async_copy(
            bufs[kk],
            out_ref.at[pl.ds(ss * _VT, _VT), :],
            sems[kk],
        )

    k = lax.rem(s, _NBUF)
    for kk in range(_NBUF):
        @pl.when(k == kk)
        def _():
            @pl.when(s >= _NBUF)
            def _():
                out_copy(kk, s).wait()

            bufs[kk][...] = lax.dot_general(
                t, a, (((1,), (1,)), ((), ())),
                preferred_element_type=jnp.float32,
            )
            out_copy(kk, s).start()

    @pl.when(s == _NV - 1)
    def _():
        for kk in range(_NBUF):
            out_copy(kk, 0).wait()


def _tc_matmul_t(avg, tag_W):
    return pl.pallas_call(
        _tc_body,
        grid=(_NV,),
        in_specs=[
            pl.BlockSpec((BATCH, EMB), lambda v: (0, 0)),
            pl.BlockSpec((_VT, EMB), lambda v: (v, 0)),
        ],
        out_specs=pl.BlockSpec(memory_space=pl.ANY),
        out_shape=jax.ShapeDtypeStruct((VOCAB, BATCH), jnp.float32),
        scratch_shapes=[pltpu.VMEM((_VT, BATCH), jnp.float32)] * _NBUF
        + [pltpu.SemaphoreType.DMA] * _NBUF,
    )(avg, tag_W)


def kernel(bags, bag_W, tag_W):
    bags_flat = bags.astype(jnp.int32).reshape(BATCH * CTX)
    bag_W_pad = jnp.pad(bag_W, ((0, 0), (0, EMB_PAD - EMB)))
    avg = _sc_bag_mean(bags_flat, bag_W_pad)
    return _tc_matmul_t(avg, tag_W).T


# speedup vs baseline: 1.0259x; 1.0259x over previous
import functools

import jax
import jax.numpy as jnp
from jax import lax
from jax.experimental import pallas as pl
from jax.experimental.pallas import tpu as pltpu
from jax.experimental.pallas import tpu_sc as plsc

VOCAB = 100000
EMB = 64
BATCH = 4096
CTX = 20

NUM_CORES = 2
NUM_SUBCORES = 16
NW = NUM_CORES * NUM_SUBCORES
BAGS_PER_W = BATCH // NW
CB = 16
NCHUNK = BAGS_PER_W // CB
ROWS_PER_CHUNK = CB * CTX
LANES = 16
EMB_PAD = 128


def _sc_bag_mean(bags_flat, bag_W_pad):
    mesh = plsc.VectorSubcoreMesh(core_axis_name="c", subcore_axis_name="s")

    @functools.partial(
        pl.kernel,
        mesh=mesh,
        out_type=jax.ShapeDtypeStruct((BATCH, EMB), jnp.float32),
        scratch_types=[
            pltpu.VMEM((BAGS_PER_W * CTX,), jnp.int32),
            pltpu.VMEM((ROWS_PER_CHUNK, EMB_PAD), jnp.float32),
            pltpu.VMEM((ROWS_PER_CHUNK, EMB_PAD), jnp.float32),
            pltpu.VMEM((BAGS_PER_W, EMB), jnp.float32),
            pltpu.SemaphoreType.DMA,
            pltpu.SemaphoreType.DMA,
        ],
    )
    def k(idx_hbm, table_hbm, out_hbm, idx_v, rows0, rows1, acc_v, sem0, sem1):
        wid = lax.axis_index("s") * NUM_CORES + lax.axis_index("c")
        pltpu.sync_copy(
            idx_hbm.at[pl.ds(wid * (BAGS_PER_W * CTX), BAGS_PER_W * CTX)],
            idx_v,
        )
        rows = (rows0, rows1)
        sems = (sem0, sem1)

        def gather(i, buf, sem):
            return pltpu.make_async_copy(
                table_hbm.at[idx_v.at[pl.ds(i * ROWS_PER_CHUNK, ROWS_PER_CHUNK)]],
                buf,
                sem,
            )

        def reduce(i, buf):
            @pl.loop(0, CB)
            def _(w):
                row0 = w * CTX
                for c in range(0, EMB, LANES):
                    s = buf[pl.ds(row0, 1), pl.ds(c, LANES)]
                    for r in range(1, CTX):
                        s = s + buf[pl.ds(row0 + r, 1), pl.ds(c, LANES)]
                    acc_v[pl.ds(i * CB + w, 1), pl.ds(c, LANES)] = s * (1.0 / CTX)

        gather(0, rows[0], sems[0]).start()
        for i in range(NCHUNK):
            b = i % 2
            gather(i, rows[b], sems[b]).wait()
            if i + 1 < NCHUNK:
                gather(i + 1, rows[1 - b], sems[1 - b]).start()
            reduce(i, rows[b])

        pltpu.sync_copy(acc_v, out_hbm.at[pl.ds(wid * BAGS_PER_W, BAGS_PER_W)])

    return k(bags_flat, bag_W_pad)


_VT = 800
_NV = VOCAB // _VT
_NBUF = 4


def _tc_body(avg_ref, tag_ref, out_ref, *scratch):
    bufs = scratch[:_NBUF]
    sems = scratch[_NBUF:]
    s = pl.program_id(0)

    a = avg_ref[...].astype(jnp.bfloat16)
    t = tag_ref[...].astype(jnp.bfloat16)

    def out_copy(kk, ss):
        return pltpu.make_async_copy(
            bufs[kk],
            out_ref.at[pl.ds(ss * _VT, _VT), :],
            sems[kk],
        )

    k = lax.rem(s, _NBUF)
    for kk in range(_NBUF):
        @pl.when(k == kk)
        def _():
            @pl.when(s >= _NBUF)
            def _():
                out_copy(kk, s).wait()

            bufs[kk][...] = lax.dot_general(
                t, a, (((1,), (1,)), ((), ())),
                preferred_element_type=jnp.float32,
            )
            out_copy(kk, s).start()

    @pl.when(s == _NV - 1)
    def _():
        for kk in range(_NBUF):
            out_copy(kk, 0).wait()


def _tc_matmul_t(avg, tag_W):
    return pl.pallas_call(
        _tc_body,
        grid=(_NV,),
        in_specs=[
            pl.BlockSpec((BATCH, EMB), lambda v: (0, 0)),
            pl.BlockSpec((_VT, EMB), lambda v: (v, 0)),
        ],
        out_specs=pl.BlockSpec(memory_space=pl.ANY),
        out_shape=jax.ShapeDtypeStruct((VOCAB, BATCH), jnp.float32),
        scratch_shapes=[pltpu.VMEM((_VT, BATCH), jnp.float32)] * _NBUF
        + [pltpu.SemaphoreType.DMA] * _NBUF,
    )(avg, tag_W)


def kernel(bags, bag_W, tag_W):
    bags_flat = bags.astype(jnp.int32).reshape(BATCH * CTX)
    bag_W_pad = jnp.pad(bag_W, ((0, 0), (0, EMB_PAD - EMB)))
    avg = _sc_bag_mean(bags_flat, bag_W_pad)
    return _tc_matmul_t(avg, tag_W).T
